# Initial kernel scaffold; baseline (speedup 1.0000x reference)
import functools

import jax
import jax.numpy as jnp
from jax import lax
from jax.experimental import pallas as pl
from jax.experimental.pallas import tpu as pltpu
from jax.experimental.pallas import tpu_sc as plsc

N_L = 4096
N_M = 16384
F1 = 64
F2 = 64
H = 128
_EPS_SQ = 1e-12
_BN_EPS = 1e-5

_QB = 256
_NQ = N_M // _QB
_MB = 512
_NB = N_M // _MB

_NW = 32
_QPW = N_M // _NW
_QCH = 256
_NCH = _QPW // _QCH
_RCH = 3 * _QCH


def _knn_body(q_ref, pt_ref, idx_ref, w_ref):
    q = q_ref[...]
    pt = pt_ref[...]
    pn = jnp.sum(pt * pt, axis=0, keepdims=True)
    qn = jnp.sum(q * q, axis=1, keepdims=True)
    s = pn - 2.0 * jnp.dot(q, pt, preferred_element_type=jnp.float32) + qn
    cols = lax.broadcasted_iota(jnp.int32, s.shape, 1)
    ms, idxs = [], []
    for _ in range(3):
        m = jnp.min(s, axis=1, keepdims=True)
        i = jnp.min(jnp.where(s == m, cols, jnp.int32(2**30)),
                    axis=1, keepdims=True)
        s = jnp.where(cols == i, jnp.float32(jnp.inf), s)
        ms.append(m)
        idxs.append(i)
    w = [1.0 / jnp.maximum(m, jnp.float32(_EPS_SQ)) for m in ms]
    inv = 1.0 / (w[0] + w[1] + w[2])
    w_ref[...] = jnp.concatenate([wk * inv for wk in w], axis=1)
    idx_ref[...] = jnp.concatenate(idxs, axis=1)


def _knn(orig_coords, prop_t):
    return pl.pallas_call(
        _knn_body,
        grid=(_NQ,),
        in_specs=[pl.BlockSpec((_QB, 3), lambda i: (i, 0)),
                  pl.BlockSpec((3, N_L), lambda i: (0, 0))],
        out_specs=[pl.BlockSpec((_QB, 3), lambda i: (i, 0)),
                   pl.BlockSpec((_QB, 3), lambda i: (i, 0))],
        out_shape=[jax.ShapeDtypeStruct((N_M, 3), jnp.int32),
                   jax.ShapeDtypeStruct((N_M, 3), jnp.float32)],
    )(orig_coords, prop_t)


def _sc_interp(prop_feats, idx_flat, w_flat):
    mesh = plsc.VectorSubcoreMesh(core_axis_name="c", subcore_axis_name="s")

    @functools.partial(
        pl.kernel,
        mesh=mesh,
        out_type=jax.ShapeDtypeStruct((N_M, F1), jnp.float32),
        scratch_types=[
            pltpu.VMEM((_RCH,), jnp.int32),
            pltpu.VMEM((_RCH,), jnp.float32),
            pltpu.VMEM((_RCH, F1), jnp.float32),
            pltpu.VMEM((_QCH, F1), jnp.float32),
            pltpu.SemaphoreType.DMA,
        ],
    )
    def k(feats_hbm, idx_hbm, wgt_hbm, out_hbm, idx_v, w_v, rows_v, out_v, sem):
        wid = lax.axis_index("s") * 2 + lax.axis_index("c")
        for ch in range(_NCH):
            qbase = wid * _QPW + ch * _QCH
            rbase = 3 * qbase
            pltpu.sync_copy(idx_hbm.at[pl.ds(rbase, _RCH)], idx_v)
            pltpu.sync_copy(wgt_hbm.at[pl.ds(rbase, _RCH)], w_v)
            pltpu.async_copy(feats_hbm.at[idx_v], rows_v, sem).wait()

            def body(qq, carry):
                r = 3 * qq
                w0 = w_v[r]
                w1 = w_v[r + 1]
                w2 = w_v[r + 2]
                for cch in range(F1 // 16):
                    slc = pl.ds(cch * 16, 16)
                    out_v[qq, slc] = (rows_v[r, slc] * w0
                                      + rows_v[r + 1, slc] * w1
                                      + rows_v[r + 2, slc] * w2)
                return carry

            lax.fori_loop(0, _QCH, body, 0)
            pltpu.sync_copy(out_v, out_hbm.at[pl.ds(qbase, _QCH)])

    return k(prop_feats, idx_flat, w_flat)


def _mlp0_body(c_ref, f_ref, it_ref, w0c_ref, w0f_ref, w0i_ref, b0_ref,
               h_ref, st_ref, acc_ref):
    h = (jnp.dot(c_ref[...], w0c_ref[...], preferred_element_type=jnp.float32)
         + jnp.dot(f_ref[...], w0f_ref[...], preferred_element_type=jnp.float32)
         + jnp.dot(it_ref[...], w0i_ref[...], preferred_element_type=jnp.float32)
         + b0_ref[...])
    h_ref[...] = h
    step = pl.program_id(0)

    @pl.when(step == 0)
    def _():
        acc_ref[...] = jnp.zeros_like(acc_ref)

    acc_ref[...] += jnp.concatenate(
        [jnp.sum(h, axis=0, keepdims=True),
         jnp.sum(h * h, axis=0, keepdims=True)], axis=0)

    @pl.when(step == _NB - 1)
    def _():
        st_ref[...] = acc_ref[...]


def _mlp0(coords, feats, interp, w0c, w0f, w0i, b0):
    return pl.pallas_call(
        _mlp0_body,
        grid=(_NB,),
        in_specs=[pl.BlockSpec((_MB, 3), lambda i: (i, 0)),
                  pl.BlockSpec((_MB, F2), lambda i: (i, 0)),
                  pl.BlockSpec((_MB, F1), lambda i: (i, 0)),
                  pl.BlockSpec((3, H), lambda i: (0, 0)),
                  pl.BlockSpec((F2, H), lambda i: (0, 0)),
                  pl.BlockSpec((F1, H), lambda i: (0, 0)),
                  pl.BlockSpec((1, H), lambda i: (0, 0))],
        out_specs=[pl.BlockSpec((_MB, H), lambda i: (i, 0)),
                   pl.BlockSpec((2, H), lambda i: (0, 0))],
        out_shape=[jax.ShapeDtypeStruct((N_M, H), jnp.float32),
                   jax.ShapeDtypeStruct((2, H), jnp.float32)],
        scratch_shapes=[pltpu.VMEM((2, H), jnp.float32)],
        compiler_params=pltpu.CompilerParams(
            dimension_semantics=("arbitrary",)),
    )(coords, feats, interp, w0c, w0f, w0i, b0)


def _bn(h, st, g, be):
    mu = st[0:1, :] * (1.0 / N_M)
    var = st[1:2, :] * (1.0 / N_M) - mu * mu
    inv = lax.rsqrt(var + jnp.float32(_BN_EPS))
    return jnp.maximum((h - mu) * inv * g + be, 0.0)


def _mlp1_body(h_ref, st_ref, g_ref, be_ref, w1_ref, b1_ref,
               h1_ref, st1_ref, acc_ref):
    z = _bn(h_ref[...], st_ref[...], g_ref[...], be_ref[...])
    h1 = jnp.dot(z, w1_ref[...], preferred_element_type=jnp.float32) + b1_ref[...]
    h1_ref[...] = h1
    step = pl.program_id(0)

    @pl.when(step == 0)
    def _():
        acc_ref[...] = jnp.zeros_like(acc_ref)

    acc_ref[...] += jnp.concatenate(
        [jnp.sum(h1, axis=0, keepdims=True),
         jnp.sum(h1 * h1, axis=0, keepdims=True)], axis=0)

    @pl.when(step == _NB - 1)
    def _():
        st1_ref[...] = acc_ref[...]


def _mlp1(h0, st0, g0, be0, W1, b1):
    return pl.pallas_call(
        _mlp1_body,
        grid=(_NB,),
        in_specs=[pl.BlockSpec((_MB, H), lambda i: (i, 0)),
                  pl.BlockSpec((2, H), lambda i: (0, 0)),
                  pl.BlockSpec((1, H), lambda i: (0, 0)),
                  pl.BlockSpec((1, H), lambda i: (0, 0)),
                  pl.BlockSpec((H, H), lambda i: (0, 0)),
                  pl.BlockSpec((1, H), lambda i: (0, 0))],
        out_specs=[pl.BlockSpec((_MB, H), lambda i: (i, 0)),
                   pl.BlockSpec((2, H), lambda i: (0, 0))],
        out_shape=[jax.ShapeDtypeStruct((N_M, H), jnp.float32),
                   jax.ShapeDtypeStruct((2, H), jnp.float32)],
        scratch_shapes=[pltpu.VMEM((2, H), jnp.float32)],
        compiler_params=pltpu.CompilerParams(
            dimension_semantics=("arbitrary",)),
    )(h0, st0, g0, be0, W1, b1)


def _bn_relu_body(h_ref, st_ref, g_ref, be_ref, o_ref):
    o_ref[...] = _bn(h_ref[...], st_ref[...], g_ref[...], be_ref[...])


def _bn_relu(h1, st1, g1, be1):
    return pl.pallas_call(
        _bn_relu_body,
        grid=(_NB,),
        in_specs=[pl.BlockSpec((_MB, H), lambda i: (i, 0)),
                  pl.BlockSpec((2, H), lambda i: (0, 0)),
                  pl.BlockSpec((1, H), lambda i: (0, 0)),
                  pl.BlockSpec((1, H), lambda i: (0, 0))],
        out_specs=pl.BlockSpec((_MB, H), lambda i: (i, 0)),
        out_shape=jax.ShapeDtypeStruct((N_M, H), jnp.float32),
    )(h1, st1, g1, be1)


def kernel(prop_coords, prop_feats, orig_coords, orig_feats,
           W0, b0, g0, be0, W1, b1, g1, be1):
    prop_t = prop_coords.T
    idx, w = _knn(orig_coords, prop_t)
    interp = _sc_interp(prop_feats, idx.reshape(-1), w.reshape(-1))
    h0, st0 = _mlp0(orig_coords, orig_feats, interp,
                    W0[0:3], W0[3:3 + F2], W0[3 + F2:],
                    b0.reshape(1, H))
    h1, st1 = _mlp1(h0, st0, g0.reshape(1, H), be0.reshape(1, H),
                    W1, b1.reshape(1, H))
    return _bn_relu(h1, st1, g1.reshape(1, H), be1.reshape(1, H))


# speedup vs baseline: 43.0635x; 43.0635x over previous
import functools

import jax
import jax.numpy as jnp
from jax import lax
from jax.experimental import pallas as pl
from jax.experimental.pallas import tpu as pltpu
from jax.experimental.pallas import tpu_sc as plsc

N_L = 4096
N_M = 16384
F1 = 64
F2 = 64
H = 128
_EPS_SQ = 1e-12
_BN_EPS = 1e-5

_QB = 256
_NQ = N_M // _QB
_MB = 512
_NB = N_M // _MB

_NW = 32
_QPW = N_M // _NW
_QCH = 256
_NCH = _QPW // _QCH
_RCH = 3 * _QCH


def _knn_body(q_ref, pt_ref, idx_ref, w_ref):
    q = q_ref[...]
    pt = pt_ref[...]
    d0 = q[:, 0:1] - pt[0:1, :]
    d1 = q[:, 1:2] - pt[1:2, :]
    d2 = q[:, 2:3] - pt[2:3, :]
    s = d0 * d0 + d1 * d1 + d2 * d2
    cols = lax.broadcasted_iota(jnp.int32, s.shape, 1)
    ms, idxs = [], []
    for _ in range(3):
        m = jnp.min(s, axis=1, keepdims=True)
        i = jnp.min(jnp.where(s == m, cols, jnp.int32(2**30)),
                    axis=1, keepdims=True)
        s = jnp.where(cols == i, jnp.float32(jnp.inf), s)
        ms.append(m)
        idxs.append(i)
    w = [1.0 / jnp.maximum(m, jnp.float32(_EPS_SQ)) for m in ms]
    inv = 1.0 / (w[0] + w[1] + w[2])
    w_ref[...] = jnp.concatenate([wk * inv for wk in w], axis=1)
    idx_ref[...] = jnp.concatenate(idxs, axis=1)


def _knn(orig_coords, prop_t):
    return pl.pallas_call(
        _knn_body,
        grid=(_NQ,),
        in_specs=[pl.BlockSpec((_QB, 3), lambda i: (i, 0)),
                  pl.BlockSpec((3, N_L), lambda i: (0, 0))],
        out_specs=[pl.BlockSpec((_QB, 3), lambda i: (i, 0)),
                   pl.BlockSpec((_QB, 3), lambda i: (i, 0))],
        out_shape=[jax.ShapeDtypeStruct((N_M, 3), jnp.int32),
                   jax.ShapeDtypeStruct((N_M, 3), jnp.float32)],
    )(orig_coords, prop_t)


def _sc_interp(prop_feats, idx_flat, w_flat):
    mesh = plsc.VectorSubcoreMesh(core_axis_name="c", subcore_axis_name="s")

    @functools.partial(
        pl.kernel,
        mesh=mesh,
        out_type=jax.ShapeDtypeStruct((N_M, F1), jnp.float32),
        scratch_types=[
            pltpu.VMEM((_RCH,), jnp.int32),
            pltpu.VMEM((_RCH + 16,), jnp.float32),
            pltpu.VMEM((_RCH, F1), jnp.float32),
            pltpu.VMEM((_QCH, F1), jnp.float32),
            pltpu.SemaphoreType.DMA,
        ],
        compiler_params=pltpu.CompilerParams(use_tc_tiling_on_sc=False),
    )
    def k(feats_hbm, idx_hbm, wgt_hbm, out_hbm, idx_v, w_v, rows_v, out_v, sem):
        wid = lax.axis_index("s") * 2 + lax.axis_index("c")
        for ch in range(_NCH):
            qbase = wid * _QPW + ch * _QCH
            rbase = 3 * qbase
            pltpu.sync_copy(idx_hbm.at[pl.ds(rbase, _RCH)], idx_v)
            pltpu.sync_copy(wgt_hbm.at[pl.ds(rbase, _RCH)],
                            w_v.at[pl.ds(0, _RCH)])
            pltpu.async_copy(feats_hbm.at[idx_v], rows_v, sem).wait()

            def body(qq, carry):
                r = 3 * qq
                wv = w_v[pl.ds(r, 16)]
                w0 = wv[0]
                w1 = wv[1]
                w2 = wv[2]
                for cch in range(F1 // 16):
                    slc = pl.ds(cch * 16, 16)
                    out_v[qq, slc] = (rows_v[r, slc] * w0
                                      + rows_v[r + 1, slc] * w1
                                      + rows_v[r + 2, slc] * w2)
                return carry

            lax.fori_loop(0, _QCH, body, 0)
            pltpu.sync_copy(out_v, out_hbm.at[pl.ds(qbase, _QCH)])

    return k(prop_feats, idx_flat, w_flat)


def _mlp0_body(c_ref, f_ref, it_ref, w0c_ref, w0f_ref, w0i_ref, b0_ref,
               h_ref, st_ref, acc_ref):
    h = (jnp.dot(c_ref[...], w0c_ref[...], preferred_element_type=jnp.float32, precision=lax.Precision.HIGHEST)
         + jnp.dot(f_ref[...], w0f_ref[...], preferred_element_type=jnp.float32, precision=lax.Precision.HIGHEST)
         + jnp.dot(it_ref[...], w0i_ref[...], preferred_element_type=jnp.float32, precision=lax.Precision.HIGHEST)
         + b0_ref[...])
    h_ref[...] = h
    step = pl.program_id(0)

    @pl.when(step == 0)
    def _():
        acc_ref[...] = jnp.zeros_like(acc_ref)

    acc_ref[...] += jnp.concatenate(
        [jnp.sum(h, axis=0, keepdims=True),
         jnp.sum(h * h, axis=0, keepdims=True)], axis=0)

    @pl.when(step == _NB - 1)
    def _():
        st_ref[...] = acc_ref[...]


def _mlp0(coords, feats, interp, w0c, w0f, w0i, b0):
    return pl.pallas_call(
        _mlp0_body,
        grid=(_NB,),
        in_specs=[pl.BlockSpec((_MB, 3), lambda i: (i, 0)),
                  pl.BlockSpec((_MB, F2), lambda i: (i, 0)),
                  pl.BlockSpec((_MB, F1), lambda i: (i, 0)),
                  pl.BlockSpec((3, H), lambda i: (0, 0)),
                  pl.BlockSpec((F2, H), lambda i: (0, 0)),
                  pl.BlockSpec((F1, H), lambda i: (0, 0)),
                  pl.BlockSpec((1, H), lambda i: (0, 0))],
        out_specs=[pl.BlockSpec((_MB, H), lambda i: (i, 0)),
                   pl.BlockSpec((2, H), lambda i: (0, 0))],
        out_shape=[jax.ShapeDtypeStruct((N_M, H), jnp.float32),
                   jax.ShapeDtypeStruct((2, H), jnp.float32)],
        scratch_shapes=[pltpu.VMEM((2, H), jnp.float32)],
        compiler_params=pltpu.CompilerParams(
            dimension_semantics=("arbitrary",)),
    )(coords, feats, interp, w0c, w0f, w0i, b0)


def _bn(h, st, g, be):
    mu = st[0:1, :] * (1.0 / N_M)
    var = st[1:2, :] * (1.0 / N_M) - mu * mu
    inv = lax.rsqrt(var + jnp.float32(_BN_EPS))
    return jnp.maximum((h - mu) * inv * g + be, 0.0)


def _mlp1_body(h_ref, st_ref, g_ref, be_ref, w1_ref, b1_ref,
               h1_ref, st1_ref, acc_ref):
    z = _bn(h_ref[...], st_ref[...], g_ref[...], be_ref[...])
    h1 = jnp.dot(z, w1_ref[...], preferred_element_type=jnp.float32, precision=lax.Precision.HIGHEST) + b1_ref[...]
    h1_ref[...] = h1
    step = pl.program_id(0)

    @pl.when(step == 0)
    def _():
        acc_ref[...] = jnp.zeros_like(acc_ref)

    acc_ref[...] += jnp.concatenate(
        [jnp.sum(h1, axis=0, keepdims=True),
         jnp.sum(h1 * h1, axis=0, keepdims=True)], axis=0)

    @pl.when(step == _NB - 1)
    def _():
        st1_ref[...] = acc_ref[...]


def _mlp1(h0, st0, g0, be0, W1, b1):
    return pl.pallas_call(
        _mlp1_body,
        grid=(_NB,),
        in_specs=[pl.BlockSpec((_MB, H), lambda i: (i, 0)),
                  pl.BlockSpec((2, H), lambda i: (0, 0)),
                  pl.BlockSpec((1, H), lambda i: (0, 0)),
                  pl.BlockSpec((1, H), lambda i: (0, 0)),
                  pl.BlockSpec((H, H), lambda i: (0, 0)),
                  pl.BlockSpec((1, H), lambda i: (0, 0))],
        out_specs=[pl.BlockSpec((_MB, H), lambda i: (i, 0)),
                   pl.BlockSpec((2, H), lambda i: (0, 0))],
        out_shape=[jax.ShapeDtypeStruct((N_M, H), jnp.float32),
                   jax.ShapeDtypeStruct((2, H), jnp.float32)],
        scratch_shapes=[pltpu.VMEM((2, H), jnp.float32)],
        compiler_params=pltpu.CompilerParams(
            dimension_semantics=("arbitrary",)),
    )(h0, st0, g0, be0, W1, b1)


def _bn_relu_body(h_ref, st_ref, g_ref, be_ref, o_ref):
    o_ref[...] = _bn(h_ref[...], st_ref[...], g_ref[...], be_ref[...])


def _bn_relu(h1, st1, g1, be1):
    return pl.pallas_call(
        _bn_relu_body,
        grid=(_NB,),
        in_specs=[pl.BlockSpec((_MB, H), lambda i: (i, 0)),
                  pl.BlockSpec((2, H), lambda i: (0, 0)),
                  pl.BlockSpec((1, H), lambda i: (0, 0)),
                  pl.BlockSpec((1, H), lambda i: (0, 0))],
        out_specs=pl.BlockSpec((_MB, H), lambda i: (i, 0)),
        out_shape=jax.ShapeDtypeStruct((N_M, H), jnp.float32),
    )(h1, st1, g1, be1)


def kernel(prop_coords, prop_feats, orig_coords, orig_feats,
           W0, b0, g0, be0, W1, b1, g1, be1):
    prop_t = prop_coords.T
    idx, w = _knn(orig_coords, prop_t)
    interp = _sc_interp(prop_feats, idx.reshape(-1), w.reshape(-1))
    h0, st0 = _mlp0(orig_coords, orig_feats, interp,
                    W0[0:3], W0[3:3 + F2], W0[3 + F2:],
                    b0.reshape(1, H))
    h1, st1 = _mlp1(h0, st0, g0.reshape(1, H), be0.reshape(1, H),
                    W1, b1.reshape(1, H))
    return _bn_relu(h1, st1, g1.reshape(1, H), be1.reshape(1, H))


# speedup vs baseline: 47.6042x; 1.1054x over previous
import functools

import jax
import jax.numpy as jnp
from jax import lax
from jax.experimental import pallas as pl
from jax.experimental.pallas import tpu as pltpu
from jax.experimental.pallas import tpu_sc as plsc

N_L = 4096
N_M = 16384
F1 = 64
F2 = 64
H = 128
_EPS_SQ = 1e-12
_BN_EPS = 1e-5

_QB = 256
_NQ = N_M // _QB
_MB = 512
_NB = N_M // _MB

_NW = 32
_QPW = N_M // _NW
_QCH = 256
_NCH = _QPW // _QCH
_RCH = 3 * _QCH


def _knn_body(q_ref, pt_ref, idx_ref, w_ref):
    q = q_ref[...]
    pt = pt_ref[...]
    d0 = q[:, 0:1] - pt[0:1, :]
    d1 = q[:, 1:2] - pt[1:2, :]
    d2 = q[:, 2:3] - pt[2:3, :]
    s = d0 * d0 + d1 * d1 + d2 * d2
    colsf = lax.broadcasted_iota(jnp.int32, s.shape, 1).astype(jnp.float32)
    big = jnp.float32(jnp.inf)
    m1 = jnp.min(s, axis=1, keepdims=True)
    m2 = jnp.min(jnp.where(s > m1, s, big), axis=1, keepdims=True)
    m3 = jnp.min(jnp.where(s > m2, s, big), axis=1, keepdims=True)
    ms = [m1, m2, m3]
    idxf = [jnp.min(jnp.where(s == m, colsf, big), axis=1, keepdims=True)
            for m in ms]
    w = [1.0 / jnp.maximum(m, jnp.float32(_EPS_SQ)) for m in ms]
    inv = 1.0 / (w[0] + w[1] + w[2])
    w_ref[...] = jnp.concatenate([wk * inv for wk in w], axis=1)
    idx_ref[...] = jnp.concatenate(
        [f.astype(jnp.int32) for f in idxf], axis=1)


def _knn(orig_coords, prop_t):
    return pl.pallas_call(
        _knn_body,
        grid=(_NQ,),
        in_specs=[pl.BlockSpec((_QB, 3), lambda i: (i, 0)),
                  pl.BlockSpec((3, N_L), lambda i: (0, 0))],
        out_specs=[pl.BlockSpec((_QB, 3), lambda i: (i, 0)),
                   pl.BlockSpec((_QB, 3), lambda i: (i, 0))],
        out_shape=[jax.ShapeDtypeStruct((N_M, 3), jnp.int32),
                   jax.ShapeDtypeStruct((N_M, 3), jnp.float32)],
    )(orig_coords, prop_t)


def _sc_interp(prop_feats, idx_flat, w_flat):
    mesh = plsc.VectorSubcoreMesh(core_axis_name="c", subcore_axis_name="s")

    @functools.partial(
        pl.kernel,
        mesh=mesh,
        out_type=jax.ShapeDtypeStruct((N_M, F1), jnp.float32),
        scratch_types=[
            pltpu.VMEM((_RCH,), jnp.int32),
            pltpu.VMEM((_RCH + 16,), jnp.float32),
            pltpu.VMEM((_RCH, F1), jnp.float32),
            pltpu.VMEM((_QCH, F1), jnp.float32),
            pltpu.SemaphoreType.DMA,
        ],
        compiler_params=pltpu.CompilerParams(use_tc_tiling_on_sc=False),
    )
    def k(feats_hbm, idx_hbm, wgt_hbm, out_hbm, idx_v, w_v, rows_v, out_v, sem):
        wid = lax.axis_index("s") * 2 + lax.axis_index("c")
        for ch in range(_NCH):
            qbase = wid * _QPW + ch * _QCH
            rbase = 3 * qbase
            pltpu.sync_copy(idx_hbm.at[pl.ds(rbase, _RCH)], idx_v)
            pltpu.sync_copy(wgt_hbm.at[pl.ds(rbase, _RCH)],
                            w_v.at[pl.ds(0, _RCH)])
            pltpu.async_copy(feats_hbm.at[idx_v], rows_v, sem).wait()

            def body(qq, carry):
                r = 3 * qq
                wv = w_v[pl.ds(r, 16)]
                w0 = wv[0]
                w1 = wv[1]
                w2 = wv[2]
                for cch in range(F1 // 16):
                    slc = pl.ds(cch * 16, 16)
                    out_v[qq, slc] = (rows_v[r, slc] * w0
                                      + rows_v[r + 1, slc] * w1
                                      + rows_v[r + 2, slc] * w2)
                return carry

            lax.fori_loop(0, _QCH, body, 0)
            pltpu.sync_copy(out_v, out_hbm.at[pl.ds(qbase, _QCH)])

    return k(prop_feats, idx_flat, w_flat)


def _mlp0_body(c_ref, f_ref, it_ref, w0c_ref, w0f_ref, w0i_ref, b0_ref,
               h_ref, st_ref, acc_ref):
    h = (jnp.dot(c_ref[...], w0c_ref[...], preferred_element_type=jnp.float32, precision=lax.Precision.HIGHEST)
         + jnp.dot(f_ref[...], w0f_ref[...], preferred_element_type=jnp.float32, precision=lax.Precision.HIGHEST)
         + jnp.dot(it_ref[...], w0i_ref[...], preferred_element_type=jnp.float32, precision=lax.Precision.HIGHEST)
         + b0_ref[...])
    h_ref[...] = h
    step = pl.program_id(0)

    @pl.when(step == 0)
    def _():
        acc_ref[...] = jnp.zeros_like(acc_ref)

    acc_ref[...] += jnp.concatenate(
        [jnp.sum(h, axis=0, keepdims=True),
         jnp.sum(h * h, axis=0, keepdims=True)], axis=0)

    @pl.when(step == _NB - 1)
    def _():
        st_ref[...] = acc_ref[...]


def _mlp0(coords, feats, interp, w0c, w0f, w0i, b0):
    return pl.pallas_call(
        _mlp0_body,
        grid=(_NB,),
        in_specs=[pl.BlockSpec((_MB, 3), lambda i: (i, 0)),
                  pl.BlockSpec((_MB, F2), lambda i: (i, 0)),
                  pl.BlockSpec((_MB, F1), lambda i: (i, 0)),
                  pl.BlockSpec((3, H), lambda i: (0, 0)),
                  pl.BlockSpec((F2, H), lambda i: (0, 0)),
                  pl.BlockSpec((F1, H), lambda i: (0, 0)),
                  pl.BlockSpec((1, H), lambda i: (0, 0))],
        out_specs=[pl.BlockSpec((_MB, H), lambda i: (i, 0)),
                   pl.BlockSpec((2, H), lambda i: (0, 0))],
        out_shape=[jax.ShapeDtypeStruct((N_M, H), jnp.float32),
                   jax.ShapeDtypeStruct((2, H), jnp.float32)],
        scratch_shapes=[pltpu.VMEM((2, H), jnp.float32)],
        compiler_params=pltpu.CompilerParams(
            dimension_semantics=("arbitrary",)),
    )(coords, feats, interp, w0c, w0f, w0i, b0)


def _bn(h, st, g, be):
    mu = st[0:1, :] * (1.0 / N_M)
    var = st[1:2, :] * (1.0 / N_M) - mu * mu
    inv = lax.rsqrt(var + jnp.float32(_BN_EPS))
    return jnp.maximum((h - mu) * inv * g + be, 0.0)


def _mlp1_body(h_ref, st_ref, g_ref, be_ref, w1_ref, b1_ref,
               h1_ref, st1_ref, acc_ref):
    z = _bn(h_ref[...], st_ref[...], g_ref[...], be_ref[...])
    h1 = jnp.dot(z, w1_ref[...], preferred_element_type=jnp.float32, precision=lax.Precision.HIGHEST) + b1_ref[...]
    h1_ref[...] = h1
    step = pl.program_id(0)

    @pl.when(step == 0)
    def _():
        acc_ref[...] = jnp.zeros_like(acc_ref)

    acc_ref[...] += jnp.concatenate(
        [jnp.sum(h1, axis=0, keepdims=True),
         jnp.sum(h1 * h1, axis=0, keepdims=True)], axis=0)

    @pl.when(step == _NB - 1)
    def _():
        st1_ref[...] = acc_ref[...]


def _mlp1(h0, st0, g0, be0, W1, b1):
    return pl.pallas_call(
        _mlp1_body,
        grid=(_NB,),
        in_specs=[pl.BlockSpec((_MB, H), lambda i: (i, 0)),
                  pl.BlockSpec((2, H), lambda i: (0, 0)),
                  pl.BlockSpec((1, H), lambda i: (0, 0)),
                  pl.BlockSpec((1, H), lambda i: (0, 0)),
                  pl.BlockSpec((H, H), lambda i: (0, 0)),
                  pl.BlockSpec((1, H), lambda i: (0, 0))],
        out_specs=[pl.BlockSpec((_MB, H), lambda i: (i, 0)),
                   pl.BlockSpec((2, H), lambda i: (0, 0))],
        out_shape=[jax.ShapeDtypeStruct((N_M, H), jnp.float32),
                   jax.ShapeDtypeStruct((2, H), jnp.float32)],
        scratch_shapes=[pltpu.VMEM((2, H), jnp.float32)],
        compiler_params=pltpu.CompilerParams(
            dimension_semantics=("arbitrary",)),
    )(h0, st0, g0, be0, W1, b1)


def _bn_relu_body(h_ref, st_ref, g_ref, be_ref, o_ref):
    o_ref[...] = _bn(h_ref[...], st_ref[...], g_ref[...], be_ref[...])


def _bn_relu(h1, st1, g1, be1):
    return pl.pallas_call(
        _bn_relu_body,
        grid=(_NB,),
        in_specs=[pl.BlockSpec((_MB, H), lambda i: (i, 0)),
                  pl.BlockSpec((2, H), lambda i: (0, 0)),
                  pl.BlockSpec((1, H), lambda i: (0, 0)),
                  pl.BlockSpec((1, H), lambda i: (0, 0))],
        out_specs=pl.BlockSpec((_MB, H), lambda i: (i, 0)),
        out_shape=jax.ShapeDtypeStruct((N_M, H), jnp.float32),
    )(h1, st1, g1, be1)


def kernel(prop_coords, prop_feats, orig_coords, orig_feats,
           W0, b0, g0, be0, W1, b1, g1, be1):
    prop_t = prop_coords.T
    idx, w = _knn(orig_coords, prop_t)
    interp = _sc_interp(prop_feats, idx.reshape(-1), w.reshape(-1))
    h0, st0 = _mlp0(orig_coords, orig_feats, interp,
                    W0[0:3], W0[3:3 + F2], W0[3 + F2:],
                    b0.reshape(1, H))
    h1, st1 = _mlp1(h0, st0, g0.reshape(1, H), be0.reshape(1, H),
                    W1, b1.reshape(1, H))
    return _bn_relu(h1, st1, g1.reshape(1, H), be1.reshape(1, H))


# speedup vs baseline: 48.9834x; 1.0290x over previous
import functools

import jax
import jax.numpy as jnp
from jax import lax
from jax.experimental import pallas as pl
from jax.experimental.pallas import tpu as pltpu
from jax.experimental.pallas import tpu_sc as plsc

N_L = 4096
N_M = 16384
F1 = 64
F2 = 64
H = 128
_EPS_SQ = 1e-12
_BN_EPS = 1e-5

_QB = 256
_NQ = N_M // _QB
_MB = 512
_NB = N_M // _MB

_NW = 32
_QPW = N_M // _NW
_QCH = 256
_NCH = _QPW // _QCH
_RCH = 3 * _QCH


def _knn_body(q_ref, pt_ref, idx_ref, w_ref):
    q = q_ref[...]
    pt = pt_ref[...]
    d0 = q[:, 0:1] - pt[0:1, :]
    d1 = q[:, 1:2] - pt[1:2, :]
    d2 = q[:, 2:3] - pt[2:3, :]
    s = d0 * d0 + d1 * d1 + d2 * d2
    colsf = lax.broadcasted_iota(jnp.int32, s.shape, 1).astype(jnp.float32)
    big = jnp.float32(jnp.inf)
    m1 = jnp.min(s, axis=1, keepdims=True)
    m2 = jnp.min(jnp.where(s > m1, s, big), axis=1, keepdims=True)
    m3 = jnp.min(jnp.where(s > m2, s, big), axis=1, keepdims=True)
    ms = [m1, m2, m3]
    idxf = [jnp.min(jnp.where(s == m, colsf, big), axis=1, keepdims=True)
            for m in ms]
    w = [1.0 / jnp.maximum(m, jnp.float32(_EPS_SQ)) for m in ms]
    inv = 1.0 / (w[0] + w[1] + w[2])
    w_ref[...] = jnp.concatenate([wk * inv for wk in w], axis=1)
    idx_ref[...] = jnp.concatenate(
        [f.astype(jnp.int32) for f in idxf], axis=1)


def _knn(orig_coords, prop_t):
    return pl.pallas_call(
        _knn_body,
        grid=(_NQ,),
        in_specs=[pl.BlockSpec((_QB, 3), lambda i: (i, 0)),
                  pl.BlockSpec((3, N_L), lambda i: (0, 0))],
        out_specs=[pl.BlockSpec((_QB, 3), lambda i: (i, 0)),
                   pl.BlockSpec((_QB, 3), lambda i: (i, 0))],
        out_shape=[jax.ShapeDtypeStruct((N_M, 3), jnp.int32),
                   jax.ShapeDtypeStruct((N_M, 3), jnp.float32)],
    )(orig_coords, prop_t)


def _sc_interp(prop_feats, idx_flat, w_flat):
    mesh = plsc.VectorSubcoreMesh(core_axis_name="c", subcore_axis_name="s")

    @functools.partial(
        pl.kernel,
        mesh=mesh,
        out_type=jax.ShapeDtypeStruct((N_M, F1), jnp.float32),
        scratch_types=[
            pltpu.VMEM((_RCH,), jnp.int32),
            pltpu.VMEM((_RCH + 16,), jnp.float32),
            pltpu.VMEM((_RCH, F1), jnp.float32),
            pltpu.VMEM((_QCH, F1), jnp.float32),
            pltpu.SemaphoreType.DMA,
        ],
        compiler_params=pltpu.CompilerParams(use_tc_tiling_on_sc=False),
    )
    def k(feats_hbm, idx_hbm, wgt_hbm, out_hbm, idx_v, w_v, rows_v, out_v, sem):
        wid = lax.axis_index("s") * 2 + lax.axis_index("c")
        for ch in range(_NCH):
            qbase = wid * _QPW + ch * _QCH
            rbase = 3 * qbase
            pltpu.sync_copy(idx_hbm.at[pl.ds(rbase, _RCH)], idx_v)
            pltpu.sync_copy(wgt_hbm.at[pl.ds(rbase, _RCH)],
                            w_v.at[pl.ds(0, _RCH)])
            pltpu.async_copy(feats_hbm.at[idx_v], rows_v, sem).wait()

            def body(qq, carry):
                r = 3 * qq
                wv = w_v[pl.ds(r, 16)]
                w0 = wv[0]
                w1 = wv[1]
                w2 = wv[2]
                for cch in range(F1 // 16):
                    slc = pl.ds(cch * 16, 16)
                    out_v[qq, slc] = (rows_v[r, slc] * w0
                                      + rows_v[r + 1, slc] * w1
                                      + rows_v[r + 2, slc] * w2)
                return carry

            lax.fori_loop(0, _QCH, body, 0)
            pltpu.sync_copy(out_v, out_hbm.at[pl.ds(qbase, _QCH)])

    return k(prop_feats, idx_flat, w_flat)


def _mlp0_body(c_ref, f_ref, it_ref, w0_ref, b0_ref,
               h_ref, st_ref, acc_ref):
    x = jnp.concatenate([c_ref[...], f_ref[...], it_ref[...]], axis=1)
    h = jnp.dot(x, w0_ref[...], preferred_element_type=jnp.float32) + b0_ref[...]
    h_ref[...] = h
    step = pl.program_id(0)

    @pl.when(step == 0)
    def _():
        acc_ref[...] = jnp.zeros_like(acc_ref)

    acc_ref[...] += jnp.concatenate(
        [jnp.sum(h, axis=0, keepdims=True),
         jnp.sum(h * h, axis=0, keepdims=True)], axis=0)

    @pl.when(step == _NB - 1)
    def _():
        st_ref[...] = acc_ref[...]


def _mlp0(coords, feats, interp, w0, b0):
    return pl.pallas_call(
        _mlp0_body,
        grid=(_NB,),
        in_specs=[pl.BlockSpec((_MB, 3), lambda i: (i, 0)),
                  pl.BlockSpec((_MB, F2), lambda i: (i, 0)),
                  pl.BlockSpec((_MB, F1), lambda i: (i, 0)),
                  pl.BlockSpec((3 + F2 + F1, H), lambda i: (0, 0)),
                  pl.BlockSpec((1, H), lambda i: (0, 0))],
        out_specs=[pl.BlockSpec((_MB, H), lambda i: (i, 0)),
                   pl.BlockSpec((2, H), lambda i: (0, 0))],
        out_shape=[jax.ShapeDtypeStruct((N_M, H), jnp.float32),
                   jax.ShapeDtypeStruct((2, H), jnp.float32)],
        scratch_shapes=[pltpu.VMEM((2, H), jnp.float32)],
        compiler_params=pltpu.CompilerParams(
            dimension_semantics=("arbitrary",)),
    )(coords, feats, interp, w0, b0)


def _bn(h, st, g, be):
    mu = st[0:1, :] * (1.0 / N_M)
    var = st[1:2, :] * (1.0 / N_M) - mu * mu
    v = var + jnp.float32(_BN_EPS)
    inv = lax.rsqrt(v)
    inv = inv * (1.5 - 0.5 * v * inv * inv)
    inv = inv * (1.5 - 0.5 * v * inv * inv)
    return jnp.maximum((h - mu) * inv * g + be, 0.0)


def _mlp1_body(h_ref, st_ref, g_ref, be_ref, w1_ref, b1_ref,
               h1_ref, st1_ref, acc_ref):
    z = _bn(h_ref[...], st_ref[...], g_ref[...], be_ref[...])
    h1 = jnp.dot(z, w1_ref[...], preferred_element_type=jnp.float32) + b1_ref[...]
    h1_ref[...] = h1
    step = pl.program_id(0)

    @pl.when(step == 0)
    def _():
        acc_ref[...] = jnp.zeros_like(acc_ref)

    acc_ref[...] += jnp.concatenate(
        [jnp.sum(h1, axis=0, keepdims=True),
         jnp.sum(h1 * h1, axis=0, keepdims=True)], axis=0)

    @pl.when(step == _NB - 1)
    def _():
        st1_ref[...] = acc_ref[...]


def _mlp1(h0, st0, g0, be0, W1, b1):
    return pl.pallas_call(
        _mlp1_body,
        grid=(_NB,),
        in_specs=[pl.BlockSpec((_MB, H), lambda i: (i, 0)),
                  pl.BlockSpec((2, H), lambda i: (0, 0)),
                  pl.BlockSpec((1, H), lambda i: (0, 0)),
                  pl.BlockSpec((1, H), lambda i: (0, 0)),
                  pl.BlockSpec((H, H), lambda i: (0, 0)),
                  pl.BlockSpec((1, H), lambda i: (0, 0))],
        out_specs=[pl.BlockSpec((_MB, H), lambda i: (i, 0)),
                   pl.BlockSpec((2, H), lambda i: (0, 0))],
        out_shape=[jax.ShapeDtypeStruct((N_M, H), jnp.float32),
                   jax.ShapeDtypeStruct((2, H), jnp.float32)],
        scratch_shapes=[pltpu.VMEM((2, H), jnp.float32)],
        compiler_params=pltpu.CompilerParams(
            dimension_semantics=("arbitrary",)),
    )(h0, st0, g0, be0, W1, b1)


def _bn_relu_body(h_ref, st_ref, g_ref, be_ref, o_ref):
    o_ref[...] = _bn(h_ref[...], st_ref[...], g_ref[...], be_ref[...])


def _bn_relu(h1, st1, g1, be1):
    return pl.pallas_call(
        _bn_relu_body,
        grid=(_NB,),
        in_specs=[pl.BlockSpec((_MB, H), lambda i: (i, 0)),
                  pl.BlockSpec((2, H), lambda i: (0, 0)),
                  pl.BlockSpec((1, H), lambda i: (0, 0)),
                  pl.BlockSpec((1, H), lambda i: (0, 0))],
        out_specs=pl.BlockSpec((_MB, H), lambda i: (i, 0)),
        out_shape=jax.ShapeDtypeStruct((N_M, H), jnp.float32),
    )(h1, st1, g1, be1)


def kernel(prop_coords, prop_feats, orig_coords, orig_feats,
           W0, b0, g0, be0, W1, b1, g1, be1):
    prop_t = prop_coords.T
    idx, w = _knn(orig_coords, prop_t)
    interp = _sc_interp(prop_feats, idx.reshape(-1), w.reshape(-1))
    h0, st0 = _mlp0(orig_coords, orig_feats, interp, W0, b0.reshape(1, H))
    h1, st1 = _mlp1(h0, st0, g0.reshape(1, H), be0.reshape(1, H),
                    W1, b1.reshape(1, H))
    return _bn_relu(h1, st1, g1.reshape(1, H), be1.reshape(1, H))
